# baseline (speedup 1.0000x reference)
import functools
import math

import jax
import jax.numpy as jnp
from jax import lax
from jax.experimental import pallas as pl
from jax.experimental.pallas import tpu as pltpu
from jax.experimental.pallas import tpu_sc as plsc

N = 10000
E = 320000
F = 128
R = 8
G = 16
C = 10
NB = 30

_BN_S = 1.0 / math.sqrt(1.0 + 1e-5)

_NC = 2
_NS = 16
_NW = _NC * _NS
_K = 80
_EPT = E // _NW
_EPC = E // _NS
_CT = 81920
_RCH = 80
_NRCH = N // _RCH
_NG = _EPT // _K
_NGC = _EPC // _K
_RD = 4

_mesh = plsc.VectorSubcoreMesh(core_axis_name="c", subcore_axis_name="s")


def _copy80(src, soff, dst):
    for j in range(_K // 16):
        dst[pl.ds(j * 16, 16)] = src[pl.ds(soff + j * 16, 16)]


@functools.partial(
    pl.kernel,
    out_type=jax.ShapeDtypeStruct((E,), jnp.float32),
    mesh=_mesh,
    scratch_types=(
        [
            pltpu.VMEM_SHARED((_CT,), jnp.float32),
            pltpu.VMEM((_CT // _NS,), jnp.float32),
            pltpu.VMEM((_EPC,), jnp.int32),
            pltpu.VMEM((_EPC,), jnp.int32),
            pltpu.VMEM((_K,), jnp.float32),
        ]
        + [pltpu.VMEM((_K,), jnp.int32)] * _RD
        + [pltpu.VMEM((_K,), jnp.float32)] * _RD
        + [pltpu.VMEM((_K,), jnp.float32)] * _RD
        + [pltpu.SemaphoreType.DMA] * (2 * _RD)
    ),
)
def _sc_edge_weights(dst_hbm, attr_hbm, w_hbm, cnt_sh, zbuf, cidx_all, tmp_all,
                     ones_b, cib0, cib1, cib2, cib3, cb0, cb1, cb2, cb3,
                     wb0, wb1, wb2, wb3, sa0, sa1, sa2, sa3, sb0, sb1, sb2, sb3):
    cib = [cib0, cib1, cib2, cib3]
    cb = [cb0, cb1, cb2, cb3]
    wb = [wb0, wb1, wb2, wb3]
    semA = [sa0, sa1, sa2, sa3]
    semB = [sb0, sb1, sb2, sb3]

    cid = lax.axis_index("c")
    sid = lax.axis_index("s")
    wid = sid * _NC + cid

    zchunk = _CT // _NS

    def _z(j, _):
        zbuf[pl.ds(j * 16, 16)] = jnp.zeros((16,), jnp.float32)
        return 0
    lax.fori_loop(0, zchunk // 16, _z, 0)
    pltpu.sync_copy(zbuf, cnt_sh.at[pl.ds(sid * zchunk, zchunk)])

    for j in range(_K // 16):
        ones_b[pl.ds(j * 16, 16)] = jnp.ones((16,), jnp.float32)

    pltpu.sync_copy(dst_hbm.at[pl.ds(sid * _EPC, _EPC)], cidx_all)
    pltpu.sync_copy(attr_hbm.at[pl.ds(sid * _EPC, _EPC)], tmp_all)

    def _mix(i, _):
        s = pl.ds(i * 16, 16)
        cidx_all[s] = cidx_all[s] * R + tmp_all[s]
        return 0
    lax.fori_loop(0, _EPC // 16, _mix, 0)

    plsc.subcore_barrier()

    def _wait_cnt(d):
        pltpu.make_async_copy(ones_b, cnt_sh.at[cib[d]], semA[d]).wait()

    def _proc_c(g, d):
        @pl.when(jnp.bool_(g >= _RD))
        def _():
            _wait_cnt(d)
        _copy80(cidx_all, g * _K, cib[d])
        pltpu.async_copy(ones_b, cnt_sh.at[cib[d]], semA[d], add=True)

    def _loop_c(s_, _):
        for d in range(_RD):
            _proc_c(s_ * _RD + d, d)
        return 0
    lax.fori_loop(0, _NGC // _RD, _loop_c, 0)
    for g in range(_NGC - _NGC % _RD, _NGC):
        _proc_c(g, g % _RD)
    for d in range(_RD):
        _wait_cnt(d)

    plsc.subcore_barrier()

    loff = cid * _EPT
    hoff = wid * _EPT

    def _fire_g(gn, d):
        _copy80(cidx_all, loff + gn * _K, cib[d])
        pltpu.async_copy(cnt_sh.at[cib[d]], cb[d], semB[d])

    def _proc_w(g, d, pd):
        pltpu.make_async_copy(cnt_sh.at[cib[d]], cb[d], semB[d]).wait()

        @pl.when(jnp.bool_(g >= _RD))
        def _():
            pltpu.make_async_copy(
                wb[d], w_hbm.at[pl.ds(hoff + (g - _RD) * _K, _K)], semA[d]).wait()
        for j in range(_K // 16):
            s = pl.ds(j * 16, 16)
            wb[d][s] = 1.0 / jnp.maximum(cb[d][s], 1.0)
        pltpu.async_copy(wb[d], w_hbm.at[pl.ds(hoff + g * _K, _K)], semA[d])

        gn = g + _RD - 1
        if isinstance(g, int):
            if gn < _NG:
                _fire_g(gn, pd)
        else:
            @pl.when(gn < _NG)
            def _():
                _fire_g(gn, pd)

    for d in range(_RD - 1):
        _fire_g(d, d)

    def _loop_w(s_, _):
        for d in range(_RD):
            _proc_w(s_ * _RD + d, d, (d - 1) % _RD)
        return 0
    lax.fori_loop(0, _NG // _RD, _loop_w, 0)
    _proc_w(_NG - 1, (_NG - 1) % _RD, (_NG - 2) % _RD)
    for k in range(_RD):
        g = _NG - _RD + k
        pltpu.make_async_copy(
            wb[g % _RD], w_hbm.at[pl.ds(hoff + g * _K, _K)], semA[g % _RD]).wait()


_SUP = 2000
_GSUP = _SUP // _K
_RDS = 3


@functools.partial(
    pl.kernel,
    out_type=jax.ShapeDtypeStruct((_NC, N, F), jnp.float32),
    mesh=_mesh,
    scratch_types=(
        [
            pltpu.VMEM_SHARED((N, F), jnp.float32),
            pltpu.VMEM((_SUP,), jnp.int32),
            pltpu.VMEM((_SUP,), jnp.int32),
            pltpu.VMEM((_SUP,), jnp.int32),
            pltpu.VMEM((_EPT,), jnp.float32),
        ]
        + [pltpu.VMEM((_K, F), jnp.float32)] * _RDS
        + [pltpu.VMEM((_K,), jnp.int32)] * _RDS
        + [pltpu.VMEM((_K,), jnp.int32)] * _RDS
        + [pltpu.SemaphoreType.DMA] * (2 * _RDS)
    ),
)
def _sc_scatter(table_hbm, src_hbm, attr_hbm, dst_hbm, w_hbm, out_hbm,
                acc_sh, sbuf, abuf, dbuf, w_all,
                r0, r1, r2, ib0, ib1, ib2, db0, db1, db2,
                sg0, sg1, sg2, ss0, ss1, ss2):
    rows = [r0, r1, r2]
    ib = [ib0, ib1, ib2]
    db = [db0, db1, db2]
    semg = [sg0, sg1, sg2]
    sems = [ss0, ss1, ss2]

    cid = lax.axis_index("c")
    sid = lax.axis_index("s")
    wid = sid * _NC + cid
    eoff = wid * _EPT
    ntrips = (_NRCH // _NS) + jnp.where(sid < (_NRCH % _NS), 1, 0)

    def _zr(r, _):
        for c8 in range(F // 16):
            rows[0][r, pl.ds(c8 * 16, 16)] = jnp.zeros((16,), jnp.float32)
        return 0
    lax.fori_loop(0, _RCH, _zr, 0)

    def _zcp(k, _):
        pltpu.sync_copy(rows[0], acc_sh.at[pl.ds((sid + k * _NS) * _RCH, _RCH)])
        return 0
    lax.fori_loop(0, ntrips, _zcp, 0)

    pltpu.sync_copy(w_hbm.at[pl.ds(eoff, _EPT)], w_all)

    def _stage(gn):
        o = eoff + gn * _K
        pltpu.sync_copy(src_hbm.at[pl.ds(o, _SUP)], sbuf)
        pltpu.sync_copy(attr_hbm.at[pl.ds(o, _SUP)], abuf)
        pltpu.sync_copy(dst_hbm.at[pl.ds(o, _SUP)], dbuf)
    _stage(0)

    plsc.subcore_barrier()

    def _fire_g(gn, d):
        lg = lax.rem(gn, _GSUP) if not isinstance(gn, int) else gn % _GSUP
        if isinstance(gn, int):
            if gn > 0 and gn % _GSUP == 0:
                _stage(gn)
        else:
            @pl.when(jnp.logical_and(lg == 0, gn > 0))
            def _():
                _stage(gn)
        lo = lg * _K
        for j in range(_K // 16):
            t = pl.ds(j * 16, 16)
            u = pl.ds(lo + j * 16, 16)
            ib[d][t] = abuf[u] * N + sbuf[u]
            db[d][t] = dbuf[u]
        pltpu.async_copy(table_hbm.at[ib[d]], rows[d], semg[d])

    def _scale(g, d):
        def _sj(j, _):
            wv = w_all[pl.ds(g * _K + j * 16, 16)]
            for l in range(16):
                we = jnp.full((16,), wv[l], jnp.float32)
                e = j * 16 + l
                for c8 in range(F // 16):
                    s = pl.ds(c8 * 16, 16)
                    rows[d][e, s] = rows[d][e, s] * we
            return 0
        lax.fori_loop(0, _K // 16, _sj, 0)

    def _proc(g, d, pd):
        pltpu.make_async_copy(table_hbm.at[ib[d]], rows[d], semg[d]).wait()
        _scale(g, d)
        pltpu.async_copy(rows[d], acc_sh.at[db[d]], sems[d], add=True)

        @pl.when(jnp.bool_(g >= 1))
        def _():
            pltpu.make_async_copy(rows[pd], acc_sh.at[db[pd]], sems[pd]).wait()

        gn = g + _RDS - 1
        if isinstance(g, int):
            if gn < _NG:
                _fire_g(gn, pd)
        else:
            @pl.when(gn < _NG)
            def _():
                _fire_g(gn, pd)

    for d in range(_RDS - 1):
        _fire_g(d, d)

    def _loop(s_, _):
        for d in range(_RDS):
            _proc(s_ * _RDS + d, d, (d - 1) % _RDS)
        return 0
    lax.fori_loop(0, _NG // _RDS, _loop, 0)
    for g in range(_NG - _NG % _RDS, _NG):
        _proc(g, g % _RDS, (g - 1) % _RDS)
    dlast = (_NG - 1) % _RDS
    pltpu.make_async_copy(rows[dlast], acc_sh.at[db[dlast]], sems[dlast]).wait()

    plsc.subcore_barrier()

    def _wb(k, _):
        rr = (sid + k * _NS) * _RCH
        pltpu.sync_copy(acc_sh.at[pl.ds(rr, _RCH)], rows[0])
        pltpu.sync_copy(rows[0], out_hbm.at[cid, pl.ds(rr, _RCH)])
        return 0
    lax.fori_loop(0, ntrips, _wb, 0)


def _wmix_body(comp_ref, basis_ref, out_ref):
    out_ref[...] = jnp.dot(comp_ref[...], basis_ref[...],
                           preferred_element_type=jnp.float32)


def _wmix(comp, basis_flat):
    return pl.pallas_call(
        _wmix_body,
        out_shape=jax.ShapeDtypeStruct((R, F * F), jnp.float32),
    )(comp, basis_flat)


_BLK = 1000
_NBLK = N // _BLK


def _layer0_body(x_ref, g_ref, b_ref, wr_ref, root_ref, bias_ref,
                 hall_ref, rrow_ref):
    r = pl.program_id(1)
    h = x_ref[...] * (g_ref[...] * _BN_S) + b_ref[...]
    hall_ref[...] = jnp.dot(h, wr_ref[0], preferred_element_type=jnp.float32)

    @pl.when(r == 0)
    def _():
        rrow_ref[...] = jnp.dot(
            h, root_ref[...], preferred_element_type=jnp.float32) + bias_ref[...]


def _layer0(x, g1, b1, wr, root, bias):
    return pl.pallas_call(
        _layer0_body,
        grid=(_NBLK, R),
        in_specs=[
            pl.BlockSpec((_BLK, F), lambda i, r: (i, 0)),
            pl.BlockSpec((1, F), lambda i, r: (0, 0)),
            pl.BlockSpec((1, F), lambda i, r: (0, 0)),
            pl.BlockSpec((1, F, F), lambda i, r: (r, 0, 0)),
            pl.BlockSpec((F, F), lambda i, r: (0, 0)),
            pl.BlockSpec((1, F), lambda i, r: (0, 0)),
        ],
        out_specs=[
            pl.BlockSpec((_BLK, F), lambda i, r: (r * _NBLK + i, 0)),
            pl.BlockSpec((_BLK, F), lambda i, r: (i, 0)),
        ],
        out_shape=[
            jax.ShapeDtypeStruct((R * N, F), jnp.float32),
            jax.ShapeDtypeStruct((N, F), jnp.float32),
        ],
    )(x, g1, b1, wr, root, bias)


def _layer1_body(p_ref, rprev_ref, wr_ref, root_ref, bias_ref,
                 hall_ref, rrow_ref):
    r = pl.program_id(1)
    h = jnp.maximum(rprev_ref[...] + p_ref[0] + p_ref[1], 0.0)
    hall_ref[...] = jnp.dot(h, wr_ref[0], preferred_element_type=jnp.float32)

    @pl.when(r == 0)
    def _():
        rrow_ref[...] = jnp.dot(
            h, root_ref[...], preferred_element_type=jnp.float32) + bias_ref[...]


def _layer1(p, rprev, wr, root, bias):
    return pl.pallas_call(
        _layer1_body,
        grid=(_NBLK, R),
        in_specs=[
            pl.BlockSpec((_NC, _BLK, F), lambda i, r: (0, i, 0)),
            pl.BlockSpec((_BLK, F), lambda i, r: (i, 0)),
            pl.BlockSpec((1, F, F), lambda i, r: (r, 0, 0)),
            pl.BlockSpec((F, F), lambda i, r: (0, 0)),
            pl.BlockSpec((1, F), lambda i, r: (0, 0)),
        ],
        out_specs=[
            pl.BlockSpec((_BLK, F), lambda i, r: (r * _NBLK + i, 0)),
            pl.BlockSpec((_BLK, F), lambda i, r: (i, 0)),
        ],
        out_shape=[
            jax.ShapeDtypeStruct((R * N, F), jnp.float32),
            jax.ShapeDtypeStruct((N, F), jnp.float32),
        ],
    )(p, rprev, wr, root, bias)


def _head_body(p_ref, rprev_ref, batch_ref, g2_ref, b2_ref,
               w1_ref, c1_ref, w2_ref, c2_ref, out_ref, sums_ref, cnts_ref):
    i = pl.program_id(0)
    h = jnp.maximum(rprev_ref[...] + p_ref[0] + p_ref[1], 0.0)
    bat = batch_ref[0]
    gid = lax.broadcasted_iota(jnp.int32, (G, _BLK), 0)
    oneh = (gid == bat).astype(jnp.float32)
    s = jnp.dot(oneh, h, preferred_element_type=jnp.float32)
    c = jnp.dot(oneh, jnp.ones((_BLK, F), jnp.float32),
                preferred_element_type=jnp.float32)

    @pl.when(i == 0)
    def _():
        sums_ref[...] = s
        cnts_ref[...] = c

    @pl.when(i > 0)
    def _():
        sums_ref[...] += s
        cnts_ref[...] += c

    @pl.when(i == _NBLK - 1)
    def _():
        mean = sums_ref[...] / jnp.maximum(cnts_ref[...], 1.0)
        hb = mean * (g2_ref[...] * _BN_S) + b2_ref[...]
        z = jnp.maximum(jnp.dot(hb, w1_ref[...],
                                preferred_element_type=jnp.float32)
                        + c1_ref[...], 0.0)
        z = jnp.dot(z, w2_ref[...],
                    preferred_element_type=jnp.float32) + c2_ref[...]
        m = jnp.max(z, axis=-1, keepdims=True)
        out_ref[...] = z - m - jnp.log(
            jnp.sum(jnp.exp(z - m), axis=-1, keepdims=True))


def _head(p, rprev, batch3d, g2, b2, w1, c1, w2, c2):
    return pl.pallas_call(
        _head_body,
        grid=(_NBLK,),
        in_specs=[
            pl.BlockSpec((_NC, _BLK, F), lambda i: (0, i, 0)),
            pl.BlockSpec((_BLK, F), lambda i: (i, 0)),
            pl.BlockSpec((1, 1, _BLK), lambda i: (i, 0, 0)),
            pl.BlockSpec((1, F), lambda i: (0, 0)),
            pl.BlockSpec((1, F), lambda i: (0, 0)),
            pl.BlockSpec((F, F), lambda i: (0, 0)),
            pl.BlockSpec((1, F), lambda i: (0, 0)),
            pl.BlockSpec((F, C), lambda i: (0, 0)),
            pl.BlockSpec((1, C), lambda i: (0, 0)),
        ],
        out_specs=pl.BlockSpec((G, C), lambda i: (0, 0)),
        out_shape=jax.ShapeDtypeStruct((G, C), jnp.float32),
        scratch_shapes=[
            pltpu.VMEM((G, F), jnp.float32),
            pltpu.VMEM((G, F), jnp.float32),
        ],
    )(p, rprev, batch3d, g2, b2, w1, c1, w2, c2)


def kernel(x, edge_index, edge_attr, batch, bn1_g, bn1_b, basis0, comp0,
           root0, bias0, basis1, comp1, root1, bias1, bn2_g, bn2_b,
           fc1_W, fc1_b, fc2_W, fc2_b):
    src = edge_index[0]
    dst = edge_index[1]

    w = _sc_edge_weights(dst, edge_attr)

    wr0 = _wmix(comp0, basis0.reshape(NB, F * F)).reshape(R, F, F)
    wr1 = _wmix(comp1, basis1.reshape(NB, F * F)).reshape(R, F, F)

    hall0, rrow0 = _layer0(x, bn1_g.reshape(1, F), bn1_b.reshape(1, F),
                           wr0, root0, bias0.reshape(1, F))
    p0 = _sc_scatter(hall0, src, edge_attr, dst, w)

    hall1, rrow1 = _layer1(p0, rrow0, wr1, root1, bias1.reshape(1, F))
    p1 = _sc_scatter(hall1, src, edge_attr, dst, w)

    return _head(p1, rrow1, batch.reshape(_NBLK, 1, _BLK),
                 bn2_g.reshape(1, F), bn2_b.reshape(1, F),
                 fc1_W, fc1_b.reshape(1, F), fc2_W, fc2_b.reshape(1, C))


# speedup vs baseline: 1.1267x; 1.1267x over previous
import functools
import math

import jax
import jax.numpy as jnp
from jax import lax
from jax.experimental import pallas as pl
from jax.experimental.pallas import tpu as pltpu
from jax.experimental.pallas import tpu_sc as plsc

N = 10000
E = 320000
F = 128
R = 8
G = 16
C = 10
NB = 30

_BN_S = 1.0 / math.sqrt(1.0 + 1e-5)

_NC = 2
_NS = 16
_NW = _NC * _NS
_K = 80
_EPT = E // _NW
_EPC = E // _NS
_CT = 81920
_RCH = 80
_NRCH = N // _RCH
_NG = _EPT // _K
_NGC = _EPC // _K
_RD = 4

_mesh = plsc.VectorSubcoreMesh(core_axis_name="c", subcore_axis_name="s")


def _copy80(src, soff, dst):
    for j in range(_K // 16):
        dst[pl.ds(j * 16, 16)] = src[pl.ds(soff + j * 16, 16)]


@functools.partial(
    pl.kernel,
    out_type=jax.ShapeDtypeStruct((E,), jnp.float32),
    mesh=_mesh,
    scratch_types=(
        [
            pltpu.VMEM_SHARED((_CT,), jnp.float32),
            pltpu.VMEM((_CT // _NS,), jnp.float32),
            pltpu.VMEM((_EPC,), jnp.int32),
            pltpu.VMEM((_EPC,), jnp.int32),
            pltpu.VMEM((_K,), jnp.float32),
        ]
        + [pltpu.VMEM((_K,), jnp.int32)] * _RD
        + [pltpu.VMEM((_K,), jnp.float32)] * _RD
        + [pltpu.VMEM((_K,), jnp.float32)] * _RD
        + [pltpu.SemaphoreType.DMA] * (2 * _RD)
    ),
)
def _sc_edge_weights(dst_hbm, attr_hbm, w_hbm, cnt_sh, zbuf, cidx_all, tmp_all,
                     ones_b, cib0, cib1, cib2, cib3, cb0, cb1, cb2, cb3,
                     wb0, wb1, wb2, wb3, sa0, sa1, sa2, sa3, sb0, sb1, sb2, sb3):
    cib = [cib0, cib1, cib2, cib3]
    cb = [cb0, cb1, cb2, cb3]
    wb = [wb0, wb1, wb2, wb3]
    semA = [sa0, sa1, sa2, sa3]
    semB = [sb0, sb1, sb2, sb3]

    cid = lax.axis_index("c")
    sid = lax.axis_index("s")
    wid = sid * _NC + cid

    zchunk = _CT // _NS

    def _z(j, _):
        zbuf[pl.ds(j * 16, 16)] = jnp.zeros((16,), jnp.float32)
        return 0
    lax.fori_loop(0, zchunk // 16, _z, 0)
    pltpu.sync_copy(zbuf, cnt_sh.at[pl.ds(sid * zchunk, zchunk)])

    for j in range(_K // 16):
        ones_b[pl.ds(j * 16, 16)] = jnp.ones((16,), jnp.float32)

    pltpu.sync_copy(dst_hbm.at[pl.ds(sid * _EPC, _EPC)], cidx_all)
    pltpu.sync_copy(attr_hbm.at[pl.ds(sid * _EPC, _EPC)], tmp_all)

    def _mix(i, _):
        s = pl.ds(i * 16, 16)
        cidx_all[s] = cidx_all[s] * R + tmp_all[s]
        return 0
    lax.fori_loop(0, _EPC // 16, _mix, 0)

    plsc.subcore_barrier()

    def _wait_cnt(d):
        pltpu.make_async_copy(ones_b, cnt_sh.at[cib[d]], semA[d]).wait()

    def _proc_c(g, d):
        @pl.when(jnp.bool_(g >= _RD))
        def _():
            _wait_cnt(d)
        _copy80(cidx_all, g * _K, cib[d])
        pltpu.async_copy(ones_b, cnt_sh.at[cib[d]], semA[d], add=True)

    def _loop_c(s_, _):
        for d in range(_RD):
            _proc_c(s_ * _RD + d, d)
        return 0
    lax.fori_loop(0, _NGC // _RD, _loop_c, 0)
    for g in range(_NGC - _NGC % _RD, _NGC):
        _proc_c(g, g % _RD)
    for d in range(_RD):
        _wait_cnt(d)

    plsc.subcore_barrier()

    loff = cid * _EPT
    hoff = wid * _EPT

    def _fire_g(gn, d):
        _copy80(cidx_all, loff + gn * _K, cib[d])
        pltpu.async_copy(cnt_sh.at[cib[d]], cb[d], semB[d])

    def _proc_w(g, d, pd):
        pltpu.make_async_copy(cnt_sh.at[cib[d]], cb[d], semB[d]).wait()

        @pl.when(jnp.bool_(g >= _RD))
        def _():
            pltpu.make_async_copy(
                wb[d], w_hbm.at[pl.ds(hoff + (g - _RD) * _K, _K)], semA[d]).wait()
        for j in range(_K // 16):
            s = pl.ds(j * 16, 16)
            wb[d][s] = 1.0 / jnp.maximum(cb[d][s], 1.0)
        pltpu.async_copy(wb[d], w_hbm.at[pl.ds(hoff + g * _K, _K)], semA[d])

        gn = g + _RD - 1
        if isinstance(g, int):
            if gn < _NG:
                _fire_g(gn, pd)
        else:
            @pl.when(gn < _NG)
            def _():
                _fire_g(gn, pd)

    for d in range(_RD - 1):
        _fire_g(d, d)

    def _loop_w(s_, _):
        for d in range(_RD):
            _proc_w(s_ * _RD + d, d, (d - 1) % _RD)
        return 0
    lax.fori_loop(0, _NG // _RD, _loop_w, 0)
    _proc_w(_NG - 1, (_NG - 1) % _RD, (_NG - 2) % _RD)
    for k in range(_RD):
        g = _NG - _RD + k
        pltpu.make_async_copy(
            wb[g % _RD], w_hbm.at[pl.ds(hoff + g * _K, _K)], semA[g % _RD]).wait()


_SUP = 2000
_GSUP = _SUP // _K
_RDS = 3


@functools.partial(
    pl.kernel,
    out_type=jax.ShapeDtypeStruct((_NC, N, F), jnp.float32),
    mesh=_mesh,
    scratch_types=(
        [
            pltpu.VMEM_SHARED((N, F), jnp.float32),
            pltpu.VMEM((_SUP,), jnp.int32),
            pltpu.VMEM((_SUP,), jnp.int32),
            pltpu.VMEM((_SUP,), jnp.int32),
            pltpu.VMEM((_EPT,), jnp.float32),
        ]
        + [pltpu.VMEM((_K, F), jnp.float32)] * _RDS
        + [pltpu.VMEM((_K,), jnp.int32)] * _RDS
        + [pltpu.VMEM((_K,), jnp.int32)] * _RDS
        + [pltpu.SemaphoreType.DMA] * (2 * _RDS)
    ),
)
def _sc_scatter(table_hbm, src_hbm, attr_hbm, dst_hbm, w_hbm, out_hbm,
                acc_sh, sbuf, abuf, dbuf, w_all,
                r0, r1, r2, ib0, ib1, ib2, db0, db1, db2,
                sg0, sg1, sg2, ss0, ss1, ss2):
    rows = [r0, r1, r2]
    ib = [ib0, ib1, ib2]
    db = [db0, db1, db2]
    semg = [sg0, sg1, sg2]
    sems = [ss0, ss1, ss2]

    cid = lax.axis_index("c")
    sid = lax.axis_index("s")
    wid = sid * _NC + cid
    eoff = wid * _EPT
    ntrips = (_NRCH // _NS) + jnp.where(sid < (_NRCH % _NS), 1, 0)

    def _zr(r, _):
        for c8 in range(F // 16):
            rows[0][r, pl.ds(c8 * 16, 16)] = jnp.zeros((16,), jnp.float32)
        return 0
    lax.fori_loop(0, _RCH, _zr, 0)

    def _zcp(k, _):
        pltpu.sync_copy(rows[0], acc_sh.at[pl.ds((sid + k * _NS) * _RCH, _RCH)])
        return 0
    lax.fori_loop(0, ntrips, _zcp, 0)

    pltpu.sync_copy(w_hbm.at[pl.ds(eoff, _EPT)], w_all)

    def _stage(gn):
        o = eoff + gn * _K
        pltpu.sync_copy(src_hbm.at[pl.ds(o, _SUP)], sbuf)
        pltpu.sync_copy(attr_hbm.at[pl.ds(o, _SUP)], abuf)
        pltpu.sync_copy(dst_hbm.at[pl.ds(o, _SUP)], dbuf)
    _stage(0)

    plsc.subcore_barrier()

    def _fire_g(gn, d):
        lg = lax.rem(gn, _GSUP) if not isinstance(gn, int) else gn % _GSUP
        if isinstance(gn, int):
            if gn > 0 and gn % _GSUP == 0:
                _stage(gn)
        else:
            @pl.when(jnp.logical_and(lg == 0, gn > 0))
            def _():
                _stage(gn)
        lo = lg * _K
        for j in range(_K // 16):
            t = pl.ds(j * 16, 16)
            u = pl.ds(lo + j * 16, 16)
            ib[d][t] = abuf[u] * N + sbuf[u]
            db[d][t] = dbuf[u]
        pltpu.async_copy(table_hbm.at[ib[d]], rows[d], semg[d])

    def _scale(g, d):
        def _sj(j, _):
            wv = w_all[pl.ds(g * _K + j * 16, 16)]
            for l in range(16):
                we = jnp.full((16,), wv[l], jnp.float32)
                e = j * 16 + l
                for c8 in range(F // 16):
                    s = pl.ds(c8 * 16, 16)
                    rows[d][e, s] = rows[d][e, s] * we
            return 0
        lax.fori_loop(0, _K // 16, _sj, 0)

    def _proc(g, d, pd):
        pltpu.make_async_copy(table_hbm.at[ib[d]], rows[d], semg[d]).wait()
        _scale(g, d)
        pltpu.async_copy(rows[d], acc_sh.at[db[d]], sems[d], add=True)

        @pl.when(jnp.bool_(g >= 1))
        def _():
            pltpu.make_async_copy(rows[pd], acc_sh.at[db[pd]], sems[pd]).wait()

        gn = g + _RDS - 1
        if isinstance(g, int):
            if gn < _NG:
                _fire_g(gn, pd)
        else:
            @pl.when(gn < _NG)
            def _():
                _fire_g(gn, pd)

    for d in range(_RDS - 1):
        _fire_g(d, d)

    def _loop(s_, _):
        for d in range(_RDS):
            _proc(s_ * _RDS + d, d, (d - 1) % _RDS)
        return 0
    lax.fori_loop(0, _NG // _RDS, _loop, 0)
    for g in range(_NG - _NG % _RDS, _NG):
        _proc(g, g % _RDS, (g - 1) % _RDS)
    dlast = (_NG - 1) % _RDS
    pltpu.make_async_copy(rows[dlast], acc_sh.at[db[dlast]], sems[dlast]).wait()

    plsc.subcore_barrier()

    def _wb(k, _):
        rr = (sid + k * _NS) * _RCH
        pltpu.sync_copy(acc_sh.at[pl.ds(rr, _RCH)], rows[0])
        pltpu.sync_copy(rows[0], out_hbm.at[cid, pl.ds(rr, _RCH)])
        return 0
    lax.fori_loop(0, ntrips, _wb, 0)


def _wmix_body(comp_ref, basis_ref, out_ref):
    out_ref[...] = jnp.dot(comp_ref[...], basis_ref[...],
                           preferred_element_type=jnp.float32)


def _wmix(comp, basis_flat):
    return pl.pallas_call(
        _wmix_body,
        out_shape=jax.ShapeDtypeStruct((R, F * F), jnp.float32),
    )(comp, basis_flat)


_BLK = 1000
_NBLK = N // _BLK
_LBLK = 2000
_LNB = N // _LBLK


def _layer0_body(x_ref, g_ref, b_ref, wr_ref, root_ref, bias_ref,
                 hall_ref, rrow_ref):
    r = pl.program_id(1)
    h = x_ref[...] * (g_ref[...] * _BN_S) + b_ref[...]
    hall_ref[...] = jnp.dot(h, wr_ref[0], preferred_element_type=jnp.float32)

    @pl.when(r == 0)
    def _():
        rrow_ref[...] = jnp.dot(
            h, root_ref[...], preferred_element_type=jnp.float32) + bias_ref[...]


def _layer0(x, g1, b1, wr, root, bias):
    return pl.pallas_call(
        _layer0_body,
        grid=(_LNB, R),
        in_specs=[
            pl.BlockSpec((_LBLK, F), lambda i, r: (i, 0)),
            pl.BlockSpec((1, F), lambda i, r: (0, 0)),
            pl.BlockSpec((1, F), lambda i, r: (0, 0)),
            pl.BlockSpec((1, F, F), lambda i, r: (r, 0, 0)),
            pl.BlockSpec((F, F), lambda i, r: (0, 0)),
            pl.BlockSpec((1, F), lambda i, r: (0, 0)),
        ],
        out_specs=[
            pl.BlockSpec((_LBLK, F), lambda i, r: (r * _LNB + i, 0)),
            pl.BlockSpec((_LBLK, F), lambda i, r: (i, 0)),
        ],
        out_shape=[
            jax.ShapeDtypeStruct((R * N, F), jnp.float32),
            jax.ShapeDtypeStruct((N, F), jnp.float32),
        ],
    )(x, g1, b1, wr, root, bias)


def _layer1_body(p_ref, rprev_ref, wr_ref, root_ref, bias_ref,
                 hall_ref, rrow_ref):
    r = pl.program_id(1)
    h = jnp.maximum(rprev_ref[...] + p_ref[0] + p_ref[1], 0.0)
    hall_ref[...] = jnp.dot(h, wr_ref[0], preferred_element_type=jnp.float32)

    @pl.when(r == 0)
    def _():
        rrow_ref[...] = jnp.dot(
            h, root_ref[...], preferred_element_type=jnp.float32) + bias_ref[...]


def _layer1(p, rprev, wr, root, bias):
    return pl.pallas_call(
        _layer1_body,
        grid=(_LNB, R),
        in_specs=[
            pl.BlockSpec((_NC, _LBLK, F), lambda i, r: (0, i, 0)),
            pl.BlockSpec((_LBLK, F), lambda i, r: (i, 0)),
            pl.BlockSpec((1, F, F), lambda i, r: (r, 0, 0)),
            pl.BlockSpec((F, F), lambda i, r: (0, 0)),
            pl.BlockSpec((1, F), lambda i, r: (0, 0)),
        ],
        out_specs=[
            pl.BlockSpec((_LBLK, F), lambda i, r: (r * _LNB + i, 0)),
            pl.BlockSpec((_LBLK, F), lambda i, r: (i, 0)),
        ],
        out_shape=[
            jax.ShapeDtypeStruct((R * N, F), jnp.float32),
            jax.ShapeDtypeStruct((N, F), jnp.float32),
        ],
    )(p, rprev, wr, root, bias)


def _head_body(p_ref, rprev_ref, batch_ref, g2_ref, b2_ref,
               w1_ref, c1_ref, w2_ref, c2_ref, out_ref, sums_ref, cnts_ref):
    i = pl.program_id(0)
    h = jnp.maximum(rprev_ref[...] + p_ref[0] + p_ref[1], 0.0)
    bat = batch_ref[0]
    gid = lax.broadcasted_iota(jnp.int32, (G, _BLK), 0)
    oneh = (gid == bat).astype(jnp.float32)
    s = jnp.dot(oneh, h, preferred_element_type=jnp.float32)
    c = jnp.dot(oneh, jnp.ones((_BLK, F), jnp.float32),
                preferred_element_type=jnp.float32)

    @pl.when(i == 0)
    def _():
        sums_ref[...] = s
        cnts_ref[...] = c

    @pl.when(i > 0)
    def _():
        sums_ref[...] += s
        cnts_ref[...] += c

    @pl.when(i == _NBLK - 1)
    def _():
        mean = sums_ref[...] / jnp.maximum(cnts_ref[...], 1.0)
        hb = mean * (g2_ref[...] * _BN_S) + b2_ref[...]
        z = jnp.maximum(jnp.dot(hb, w1_ref[...],
                                preferred_element_type=jnp.float32)
                        + c1_ref[...], 0.0)
        z = jnp.dot(z, w2_ref[...],
                    preferred_element_type=jnp.float32) + c2_ref[...]
        m = jnp.max(z, axis=-1, keepdims=True)
        out_ref[...] = z - m - jnp.log(
            jnp.sum(jnp.exp(z - m), axis=-1, keepdims=True))


def _head(p, rprev, batch3d, g2, b2, w1, c1, w2, c2):
    return pl.pallas_call(
        _head_body,
        grid=(_NBLK,),
        in_specs=[
            pl.BlockSpec((_NC, _BLK, F), lambda i: (0, i, 0)),
            pl.BlockSpec((_BLK, F), lambda i: (i, 0)),
            pl.BlockSpec((1, 1, _BLK), lambda i: (i, 0, 0)),
            pl.BlockSpec((1, F), lambda i: (0, 0)),
            pl.BlockSpec((1, F), lambda i: (0, 0)),
            pl.BlockSpec((F, F), lambda i: (0, 0)),
            pl.BlockSpec((1, F), lambda i: (0, 0)),
            pl.BlockSpec((F, C), lambda i: (0, 0)),
            pl.BlockSpec((1, C), lambda i: (0, 0)),
        ],
        out_specs=pl.BlockSpec((G, C), lambda i: (0, 0)),
        out_shape=jax.ShapeDtypeStruct((G, C), jnp.float32),
        scratch_shapes=[
            pltpu.VMEM((G, F), jnp.float32),
            pltpu.VMEM((G, F), jnp.float32),
        ],
    )(p, rprev, batch3d, g2, b2, w1, c1, w2, c2)


def kernel(x, edge_index, edge_attr, batch, bn1_g, bn1_b, basis0, comp0,
           root0, bias0, basis1, comp1, root1, bias1, bn2_g, bn2_b,
           fc1_W, fc1_b, fc2_W, fc2_b):
    src = edge_index[0]
    dst = edge_index[1]

    w = _sc_edge_weights(dst, edge_attr)

    wr0 = _wmix(comp0, basis0.reshape(NB, F * F)).reshape(R, F, F)
    wr1 = _wmix(comp1, basis1.reshape(NB, F * F)).reshape(R, F, F)

    hall0, rrow0 = _layer0(x, bn1_g.reshape(1, F), bn1_b.reshape(1, F),
                           wr0, root0, bias0.reshape(1, F))
    p0 = _sc_scatter(hall0, src, edge_attr, dst, w)

    hall1, rrow1 = _layer1(p0, rrow0, wr1, root1, bias1.reshape(1, F))
    p1 = _sc_scatter(hall1, src, edge_attr, dst, w)

    return _head(p1, rrow1, batch.reshape(_NBLK, 1, _BLK),
                 bn2_g.reshape(1, F), bn2_b.reshape(1, F),
                 fc1_W, fc1_b.reshape(1, F), fc2_W, fc2_b.reshape(1, C))


# speedup vs baseline: 1.2019x; 1.0668x over previous
import functools
import math

import jax
import jax.numpy as jnp
from jax import lax
from jax.experimental import pallas as pl
from jax.experimental.pallas import tpu as pltpu
from jax.experimental.pallas import tpu_sc as plsc

N = 10000
E = 320000
F = 128
R = 8
G = 16
C = 10
NB = 30

_BN_S = 1.0 / math.sqrt(1.0 + 1e-5)

_NC = 2
_NS = 16
_NW = _NC * _NS
_K = 80
_EPT = E // _NW
_EPC = E // _NS
_CT = 81920
_RCH = 80
_NRCH = N // _RCH
_NG = _EPT // _K
_NGC = _EPC // _K
_RD = 4

_mesh = plsc.VectorSubcoreMesh(core_axis_name="c", subcore_axis_name="s")


def _copy80(src, soff, dst):
    for j in range(_K // 16):
        dst[pl.ds(j * 16, 16)] = src[pl.ds(soff + j * 16, 16)]


@functools.partial(
    pl.kernel,
    out_type=jax.ShapeDtypeStruct((E,), jnp.float32),
    mesh=_mesh,
    scratch_types=(
        [
            pltpu.VMEM_SHARED((_CT,), jnp.float32),
            pltpu.VMEM((_CT // _NS,), jnp.float32),
            pltpu.VMEM((_EPC,), jnp.int32),
            pltpu.VMEM((_EPC,), jnp.int32),
            pltpu.VMEM((_K,), jnp.float32),
        ]
        + [pltpu.VMEM((_K,), jnp.int32)] * _RD
        + [pltpu.VMEM((_K,), jnp.float32)] * _RD
        + [pltpu.VMEM((_K,), jnp.float32)] * _RD
        + [pltpu.SemaphoreType.DMA] * (2 * _RD)
    ),
)
def _sc_edge_weights(dst_hbm, attr_hbm, w_hbm, cnt_sh, zbuf, cidx_all, tmp_all,
                     ones_b, cib0, cib1, cib2, cib3, cb0, cb1, cb2, cb3,
                     wb0, wb1, wb2, wb3, sa0, sa1, sa2, sa3, sb0, sb1, sb2, sb3):
    cib = [cib0, cib1, cib2, cib3]
    cb = [cb0, cb1, cb2, cb3]
    wb = [wb0, wb1, wb2, wb3]
    semA = [sa0, sa1, sa2, sa3]
    semB = [sb0, sb1, sb2, sb3]

    cid = lax.axis_index("c")
    sid = lax.axis_index("s")
    wid = sid * _NC + cid

    zchunk = _CT // _NS

    def _z(j, _):
        zbuf[pl.ds(j * 16, 16)] = jnp.zeros((16,), jnp.float32)
        return 0
    lax.fori_loop(0, zchunk // 16, _z, 0)
    pltpu.sync_copy(zbuf, cnt_sh.at[pl.ds(sid * zchunk, zchunk)])

    for j in range(_K // 16):
        ones_b[pl.ds(j * 16, 16)] = jnp.ones((16,), jnp.float32)

    pltpu.sync_copy(dst_hbm.at[pl.ds(sid * _EPC, _EPC)], cidx_all)
    pltpu.sync_copy(attr_hbm.at[pl.ds(sid * _EPC, _EPC)], tmp_all)

    def _mix(i, _):
        s = pl.ds(i * 16, 16)
        cidx_all[s] = cidx_all[s] * R + tmp_all[s]
        return 0
    lax.fori_loop(0, _EPC // 16, _mix, 0)

    plsc.subcore_barrier()

    def _wait_cnt(d):
        pltpu.make_async_copy(ones_b, cnt_sh.at[cib[d]], semA[d]).wait()

    def _proc_c(g, d):
        @pl.when(jnp.bool_(g >= _RD))
        def _():
            _wait_cnt(d)
        _copy80(cidx_all, g * _K, cib[d])
        pltpu.async_copy(ones_b, cnt_sh.at[cib[d]], semA[d], add=True)

    def _loop_c(s_, _):
        for d in range(_RD):
            _proc_c(s_ * _RD + d, d)
        return 0
    lax.fori_loop(0, _NGC // _RD, _loop_c, 0)
    for g in range(_NGC - _NGC % _RD, _NGC):
        _proc_c(g, g % _RD)
    for d in range(_RD):
        _wait_cnt(d)

    plsc.subcore_barrier()

    loff = cid * _EPT
    hoff = wid * _EPT

    def _fire_g(gn, d):
        _copy80(cidx_all, loff + gn * _K, cib[d])
        pltpu.async_copy(cnt_sh.at[cib[d]], cb[d], semB[d])

    def _proc_w(g, d, pd):
        pltpu.make_async_copy(cnt_sh.at[cib[d]], cb[d], semB[d]).wait()

        @pl.when(jnp.bool_(g >= _RD))
        def _():
            pltpu.make_async_copy(
                wb[d], w_hbm.at[pl.ds(hoff + (g - _RD) * _K, _K)], semA[d]).wait()
        for j in range(_K // 16):
            s = pl.ds(j * 16, 16)
            wb[d][s] = 1.0 / jnp.maximum(cb[d][s], 1.0)
        pltpu.async_copy(wb[d], w_hbm.at[pl.ds(hoff + g * _K, _K)], semA[d])

        gn = g + _RD - 1
        if isinstance(g, int):
            if gn < _NG:
                _fire_g(gn, pd)
        else:
            @pl.when(gn < _NG)
            def _():
                _fire_g(gn, pd)

    for d in range(_RD - 1):
        _fire_g(d, d)

    def _loop_w(s_, _):
        for d in range(_RD):
            _proc_w(s_ * _RD + d, d, (d - 1) % _RD)
        return 0
    lax.fori_loop(0, _NG // _RD, _loop_w, 0)
    _proc_w(_NG - 1, (_NG - 1) % _RD, (_NG - 2) % _RD)
    for k in range(_RD):
        g = _NG - _RD + k
        pltpu.make_async_copy(
            wb[g % _RD], w_hbm.at[pl.ds(hoff + g * _K, _K)], semA[g % _RD]).wait()


_SUP = 2000
_GSUP = _SUP // _K
_RDS = 3


@functools.partial(
    pl.kernel,
    out_type=jax.ShapeDtypeStruct((_NC, N, F), jnp.float32),
    mesh=_mesh,
    scratch_types=(
        [
            pltpu.VMEM_SHARED((N, F), jnp.float32),
            pltpu.VMEM((_SUP,), jnp.int32),
            pltpu.VMEM((_SUP,), jnp.int32),
            pltpu.VMEM((_SUP,), jnp.int32),
            pltpu.VMEM((_EPT,), jnp.float32),
        ]
        + [pltpu.VMEM((_K, F), jnp.float32)] * _RDS
        + [pltpu.VMEM((_K,), jnp.int32)] * _RDS
        + [pltpu.VMEM((_K,), jnp.int32)] * _RDS
        + [pltpu.SemaphoreType.DMA] * (2 * _RDS)
    ),
)
def _sc_scatter(table_hbm, src_hbm, attr_hbm, dst_hbm, w_hbm, out_hbm,
                acc_sh, sbuf, abuf, dbuf, w_all,
                r0, r1, r2, ib0, ib1, ib2, db0, db1, db2,
                sg0, sg1, sg2, ss0, ss1, ss2):
    rows = [r0, r1, r2]
    ib = [ib0, ib1, ib2]
    db = [db0, db1, db2]
    semg = [sg0, sg1, sg2]
    sems = [ss0, ss1, ss2]

    cid = lax.axis_index("c")
    sid = lax.axis_index("s")
    wid = sid * _NC + cid
    eoff = wid * _EPT
    ntrips = (_NRCH // _NS) + jnp.where(sid < (_NRCH % _NS), 1, 0)

    def _zr(r, _):
        for c8 in range(F // 16):
            rows[0][r, pl.ds(c8 * 16, 16)] = jnp.zeros((16,), jnp.float32)
        return 0
    lax.fori_loop(0, _RCH, _zr, 0)

    def _zcp(k, _):
        pltpu.sync_copy(rows[0], acc_sh.at[pl.ds((sid + k * _NS) * _RCH, _RCH)])
        return 0
    lax.fori_loop(0, ntrips, _zcp, 0)

    pltpu.sync_copy(w_hbm.at[pl.ds(eoff, _EPT)], w_all)

    def _stage(gn):
        o = eoff + gn * _K
        pltpu.sync_copy(src_hbm.at[pl.ds(o, _SUP)], sbuf)
        pltpu.sync_copy(attr_hbm.at[pl.ds(o, _SUP)], abuf)
        pltpu.sync_copy(dst_hbm.at[pl.ds(o, _SUP)], dbuf)
    _stage(0)

    plsc.subcore_barrier()

    def _fire_g(gn, d):
        lg = lax.rem(gn, _GSUP) if not isinstance(gn, int) else gn % _GSUP
        if isinstance(gn, int):
            if gn > 0 and gn % _GSUP == 0:
                _stage(gn)
        else:
            @pl.when(jnp.logical_and(lg == 0, gn > 0))
            def _():
                _stage(gn)
        lo = lg * _K
        for j in range(_K // 16):
            t = pl.ds(j * 16, 16)
            u = pl.ds(lo + j * 16, 16)
            ib[d][t] = abuf[u] * N + sbuf[u]
            db[d][t] = dbuf[u]
        pltpu.async_copy(table_hbm.at[ib[d]], rows[d], semg[d])

    def _scale(g, d):
        def _sj(j, _):
            wv = w_all[pl.ds(g * _K + j * 16, 16)]
            for l in range(16):
                we = jnp.full((16,), wv[l], jnp.float32)
                e = j * 16 + l
                for c8 in range(F // 16):
                    s = pl.ds(c8 * 16, 16)
                    rows[d][e, s] = rows[d][e, s] * we
            return 0
        lax.fori_loop(0, _K // 16, _sj, 0)

    def _proc(g, d, pd):
        pltpu.make_async_copy(table_hbm.at[ib[d]], rows[d], semg[d]).wait()
        _scale(g, d)
        pltpu.async_copy(rows[d], acc_sh.at[db[d]], sems[d], add=True)

        @pl.when(jnp.bool_(g >= 1))
        def _():
            pltpu.make_async_copy(rows[pd], acc_sh.at[db[pd]], sems[pd]).wait()

        gn = g + _RDS - 1
        if isinstance(g, int):
            if gn < _NG:
                _fire_g(gn, pd)
        else:
            @pl.when(gn < _NG)
            def _():
                _fire_g(gn, pd)

    for d in range(_RDS - 1):
        _fire_g(d, d)

    def _loop(s_, _):
        for d in range(_RDS):
            _proc(s_ * _RDS + d, d, (d - 1) % _RDS)
        return 0
    lax.fori_loop(0, _NG // _RDS, _loop, 0)
    for g in range(_NG - _NG % _RDS, _NG):
        _proc(g, g % _RDS, (g - 1) % _RDS)
    dlast = (_NG - 1) % _RDS
    pltpu.make_async_copy(rows[dlast], acc_sh.at[db[dlast]], sems[dlast]).wait()

    plsc.subcore_barrier()

    def _wb(k, _):
        rr = (sid + k * _NS) * _RCH
        pltpu.sync_copy(acc_sh.at[pl.ds(rr, _RCH)], rows[0])
        pltpu.sync_copy(rows[0], out_hbm.at[cid, pl.ds(rr, _RCH)])
        return 0
    lax.fori_loop(0, ntrips, _wb, 0)


def _wmix_body(comp_ref, basis_ref, out_ref):
    out_ref[...] = jnp.dot(comp_ref[...], basis_ref[...],
                           preferred_element_type=jnp.float32)


def _wmix(comp, basis_flat):
    return pl.pallas_call(
        _wmix_body,
        out_shape=jax.ShapeDtypeStruct((R, F * F), jnp.float32),
    )(comp, basis_flat)


_BLK = 1000
_NBLK = N // _BLK
_LBLK = 5000
_LNB = N // _LBLK


def _layer0_body(x_ref, g_ref, b_ref, wr_ref, root_ref, bias_ref,
                 hall_ref, rrow_ref):
    r = pl.program_id(1)
    h = x_ref[...] * (g_ref[...] * _BN_S) + b_ref[...]
    hall_ref[...] = jnp.dot(h, wr_ref[0], preferred_element_type=jnp.float32)

    @pl.when(r == 0)
    def _():
        rrow_ref[...] = jnp.dot(
            h, root_ref[...], preferred_element_type=jnp.float32) + bias_ref[...]


def _layer0(x, g1, b1, wr, root, bias):
    return pl.pallas_call(
        _layer0_body,
        grid=(_LNB, R),
        in_specs=[
            pl.BlockSpec((_LBLK, F), lambda i, r: (i, 0)),
            pl.BlockSpec((1, F), lambda i, r: (0, 0)),
            pl.BlockSpec((1, F), lambda i, r: (0, 0)),
            pl.BlockSpec((1, F, F), lambda i, r: (r, 0, 0)),
            pl.BlockSpec((F, F), lambda i, r: (0, 0)),
            pl.BlockSpec((1, F), lambda i, r: (0, 0)),
        ],
        out_specs=[
            pl.BlockSpec((_LBLK, F), lambda i, r: (r * _LNB + i, 0)),
            pl.BlockSpec((_LBLK, F), lambda i, r: (i, 0)),
        ],
        out_shape=[
            jax.ShapeDtypeStruct((R * N, F), jnp.float32),
            jax.ShapeDtypeStruct((N, F), jnp.float32),
        ],
    )(x, g1, b1, wr, root, bias)


def _layer1_body(p_ref, rprev_ref, wr_ref, root_ref, bias_ref,
                 hall_ref, rrow_ref):
    r = pl.program_id(1)
    h = jnp.maximum(rprev_ref[...] + p_ref[0] + p_ref[1], 0.0)
    hall_ref[...] = jnp.dot(h, wr_ref[0], preferred_element_type=jnp.float32)

    @pl.when(r == 0)
    def _():
        rrow_ref[...] = jnp.dot(
            h, root_ref[...], preferred_element_type=jnp.float32) + bias_ref[...]


def _layer1(p, rprev, wr, root, bias):
    return pl.pallas_call(
        _layer1_body,
        grid=(_LNB, R),
        in_specs=[
            pl.BlockSpec((_NC, _LBLK, F), lambda i, r: (0, i, 0)),
            pl.BlockSpec((_LBLK, F), lambda i, r: (i, 0)),
            pl.BlockSpec((1, F, F), lambda i, r: (r, 0, 0)),
            pl.BlockSpec((F, F), lambda i, r: (0, 0)),
            pl.BlockSpec((1, F), lambda i, r: (0, 0)),
        ],
        out_specs=[
            pl.BlockSpec((_LBLK, F), lambda i, r: (r * _LNB + i, 0)),
            pl.BlockSpec((_LBLK, F), lambda i, r: (i, 0)),
        ],
        out_shape=[
            jax.ShapeDtypeStruct((R * N, F), jnp.float32),
            jax.ShapeDtypeStruct((N, F), jnp.float32),
        ],
    )(p, rprev, wr, root, bias)


def _head_body(p_ref, rprev_ref, batch_ref, g2_ref, b2_ref,
               w1_ref, c1_ref, w2_ref, c2_ref, out_ref, sums_ref, cnts_ref):
    i = pl.program_id(0)
    h = jnp.maximum(rprev_ref[...] + p_ref[0] + p_ref[1], 0.0)
    bat = batch_ref[0]
    gid = lax.broadcasted_iota(jnp.int32, (G, _BLK), 0)
    oneh = (gid == bat).astype(jnp.float32)
    s = jnp.dot(oneh, h, preferred_element_type=jnp.float32)
    c = jnp.dot(oneh, jnp.ones((_BLK, F), jnp.float32),
                preferred_element_type=jnp.float32)

    @pl.when(i == 0)
    def _():
        sums_ref[...] = s
        cnts_ref[...] = c

    @pl.when(i > 0)
    def _():
        sums_ref[...] += s
        cnts_ref[...] += c

    @pl.when(i == _NBLK - 1)
    def _():
        mean = sums_ref[...] / jnp.maximum(cnts_ref[...], 1.0)
        hb = mean * (g2_ref[...] * _BN_S) + b2_ref[...]
        z = jnp.maximum(jnp.dot(hb, w1_ref[...],
                                preferred_element_type=jnp.float32)
                        + c1_ref[...], 0.0)
        z = jnp.dot(z, w2_ref[...],
                    preferred_element_type=jnp.float32) + c2_ref[...]
        m = jnp.max(z, axis=-1, keepdims=True)
        out_ref[...] = z - m - jnp.log(
            jnp.sum(jnp.exp(z - m), axis=-1, keepdims=True))


def _head(p, rprev, batch3d, g2, b2, w1, c1, w2, c2):
    return pl.pallas_call(
        _head_body,
        grid=(_NBLK,),
        in_specs=[
            pl.BlockSpec((_NC, _BLK, F), lambda i: (0, i, 0)),
            pl.BlockSpec((_BLK, F), lambda i: (i, 0)),
            pl.BlockSpec((1, 1, _BLK), lambda i: (i, 0, 0)),
            pl.BlockSpec((1, F), lambda i: (0, 0)),
            pl.BlockSpec((1, F), lambda i: (0, 0)),
            pl.BlockSpec((F, F), lambda i: (0, 0)),
            pl.BlockSpec((1, F), lambda i: (0, 0)),
            pl.BlockSpec((F, C), lambda i: (0, 0)),
            pl.BlockSpec((1, C), lambda i: (0, 0)),
        ],
        out_specs=pl.BlockSpec((G, C), lambda i: (0, 0)),
        out_shape=jax.ShapeDtypeStruct((G, C), jnp.float32),
        scratch_shapes=[
            pltpu.VMEM((G, F), jnp.float32),
            pltpu.VMEM((G, F), jnp.float32),
        ],
    )(p, rprev, batch3d, g2, b2, w1, c1, w2, c2)


def kernel(x, edge_index, edge_attr, batch, bn1_g, bn1_b, basis0, comp0,
           root0, bias0, basis1, comp1, root1, bias1, bn2_g, bn2_b,
           fc1_W, fc1_b, fc2_W, fc2_b):
    src = edge_index[0]
    dst = edge_index[1]

    w = _sc_edge_weights(dst, edge_attr)

    wr0 = _wmix(comp0, basis0.reshape(NB, F * F)).reshape(R, F, F)
    wr1 = _wmix(comp1, basis1.reshape(NB, F * F)).reshape(R, F, F)

    hall0, rrow0 = _layer0(x, bn1_g.reshape(1, F), bn1_b.reshape(1, F),
                           wr0, root0, bias0.reshape(1, F))
    p0 = _sc_scatter(hall0, src, edge_attr, dst, w)

    hall1, rrow1 = _layer1(p0, rrow0, wr1, root1, bias1.reshape(1, F))
    p1 = _sc_scatter(hall1, src, edge_attr, dst, w)

    return _head(p1, rrow1, batch.reshape(_NBLK, 1, _BLK),
                 bn2_g.reshape(1, F), bn2_b.reshape(1, F),
                 fc1_W, fc1_b.reshape(1, F), fc2_W, fc2_b.reshape(1, C))


# speedup vs baseline: 1.2458x; 1.0365x over previous
import functools
import math

import jax
import jax.numpy as jnp
from jax import lax
from jax.experimental import pallas as pl
from jax.experimental.pallas import tpu as pltpu
from jax.experimental.pallas import tpu_sc as plsc

N = 10000
E = 320000
F = 128
R = 8
G = 16
C = 10
NB = 30

_BN_S = 1.0 / math.sqrt(1.0 + 1e-5)

_NC = 2
_NS = 16
_NW = _NC * _NS
_K = 80
_EPT = E // _NW
_EPC = E // _NS
_CT = 81920
_RCH = 80
_NRCH = N // _RCH
_NG = _EPT // _K
_NGC = _EPC // _K
_RD = 4

_mesh = plsc.VectorSubcoreMesh(core_axis_name="c", subcore_axis_name="s")


def _copy80(src, soff, dst):
    for j in range(_K // 16):
        dst[pl.ds(j * 16, 16)] = src[pl.ds(soff + j * 16, 16)]


@functools.partial(
    pl.kernel,
    out_type=jax.ShapeDtypeStruct((E,), jnp.float32),
    mesh=_mesh,
    scratch_types=(
        [
            pltpu.VMEM_SHARED((_CT,), jnp.float32),
            pltpu.VMEM((_CT // _NS,), jnp.float32),
            pltpu.VMEM((_EPC,), jnp.int32),
            pltpu.VMEM((_EPC,), jnp.int32),
            pltpu.VMEM((_K,), jnp.float32),
        ]
        + [pltpu.VMEM((_K,), jnp.int32)] * _RD
        + [pltpu.VMEM((_K,), jnp.float32)] * _RD
        + [pltpu.VMEM((_K,), jnp.float32)] * _RD
        + [pltpu.SemaphoreType.DMA] * (2 * _RD)
    ),
)
def _sc_edge_weights(dst_hbm, attr_hbm, w_hbm, cnt_sh, zbuf, cidx_all, tmp_all,
                     ones_b, cib0, cib1, cib2, cib3, cb0, cb1, cb2, cb3,
                     wb0, wb1, wb2, wb3, sa0, sa1, sa2, sa3, sb0, sb1, sb2, sb3):
    cib = [cib0, cib1, cib2, cib3]
    cb = [cb0, cb1, cb2, cb3]
    wb = [wb0, wb1, wb2, wb3]
    semA = [sa0, sa1, sa2, sa3]
    semB = [sb0, sb1, sb2, sb3]

    cid = lax.axis_index("c")
    sid = lax.axis_index("s")
    wid = sid * _NC + cid

    zchunk = _CT // _NS

    def _z(j, _):
        zbuf[pl.ds(j * 16, 16)] = jnp.zeros((16,), jnp.float32)
        return 0
    lax.fori_loop(0, zchunk // 16, _z, 0)
    pltpu.sync_copy(zbuf, cnt_sh.at[pl.ds(sid * zchunk, zchunk)])

    for j in range(_K // 16):
        ones_b[pl.ds(j * 16, 16)] = jnp.ones((16,), jnp.float32)

    pltpu.sync_copy(dst_hbm.at[pl.ds(sid * _EPC, _EPC)], cidx_all)
    pltpu.sync_copy(attr_hbm.at[pl.ds(sid * _EPC, _EPC)], tmp_all)

    def _mix(i, _):
        s = pl.ds(i * 16, 16)
        cidx_all[s] = cidx_all[s] * R + tmp_all[s]
        return 0
    lax.fori_loop(0, _EPC // 16, _mix, 0)

    plsc.subcore_barrier()

    def _wait_cnt(d):
        pltpu.make_async_copy(ones_b, cnt_sh.at[cib[d]], semA[d]).wait()

    def _proc_c(g, d):
        @pl.when(jnp.bool_(g >= _RD))
        def _():
            _wait_cnt(d)
        _copy80(cidx_all, g * _K, cib[d])
        pltpu.async_copy(ones_b, cnt_sh.at[cib[d]], semA[d], add=True)

    def _loop_c(s_, _):
        for d in range(_RD):
            _proc_c(s_ * _RD + d, d)
        return 0
    lax.fori_loop(0, _NGC // _RD, _loop_c, 0)
    for g in range(_NGC - _NGC % _RD, _NGC):
        _proc_c(g, g % _RD)
    for d in range(_RD):
        _wait_cnt(d)

    plsc.subcore_barrier()

    loff = cid * _EPT
    hoff = wid * _EPT

    def _fire_g(gn, d):
        _copy80(cidx_all, loff + gn * _K, cib[d])
        pltpu.async_copy(cnt_sh.at[cib[d]], cb[d], semB[d])

    def _proc_w(g, d, pd):
        pltpu.make_async_copy(cnt_sh.at[cib[d]], cb[d], semB[d]).wait()

        @pl.when(jnp.bool_(g >= _RD))
        def _():
            pltpu.make_async_copy(
                wb[d], w_hbm.at[pl.ds(hoff + (g - _RD) * _K, _K)], semA[d]).wait()
        for j in range(_K // 16):
            s = pl.ds(j * 16, 16)
            wb[d][s] = 1.0 / jnp.maximum(cb[d][s], 1.0)
        pltpu.async_copy(wb[d], w_hbm.at[pl.ds(hoff + g * _K, _K)], semA[d])

        gn = g + _RD - 1
        if isinstance(g, int):
            if gn < _NG:
                _fire_g(gn, pd)
        else:
            @pl.when(gn < _NG)
            def _():
                _fire_g(gn, pd)

    for d in range(_RD - 1):
        _fire_g(d, d)

    def _loop_w(s_, _):
        for d in range(_RD):
            _proc_w(s_ * _RD + d, d, (d - 1) % _RD)
        return 0
    lax.fori_loop(0, _NG // _RD, _loop_w, 0)
    _proc_w(_NG - 1, (_NG - 1) % _RD, (_NG - 2) % _RD)
    for k in range(_RD):
        g = _NG - _RD + k
        pltpu.make_async_copy(
            wb[g % _RD], w_hbm.at[pl.ds(hoff + g * _K, _K)], semA[g % _RD]).wait()


_SUP = 2000
_GSUP = _SUP // _K
_RDS = 3


@functools.partial(
    pl.kernel,
    out_type=jax.ShapeDtypeStruct((_NC, N, F), jnp.float32),
    mesh=_mesh,
    scratch_types=(
        [
            pltpu.VMEM_SHARED((N, F), jnp.float32),
            pltpu.VMEM((_SUP,), jnp.int32),
            pltpu.VMEM((_SUP,), jnp.int32),
            pltpu.VMEM((_SUP,), jnp.int32),
            pltpu.VMEM((_EPT,), jnp.float32),
        ]
        + [pltpu.VMEM((_K, F), jnp.float32)] * _RDS
        + [pltpu.VMEM((_K,), jnp.int32)] * _RDS
        + [pltpu.VMEM((_K,), jnp.int32)] * _RDS
        + [pltpu.SemaphoreType.DMA] * (2 * _RDS)
    ),
)
def _sc_scatter(table_hbm, src_hbm, attr_hbm, dst_hbm, w_hbm, out_hbm,
                acc_sh, sbuf, abuf, dbuf, w_all,
                r0, r1, r2, ib0, ib1, ib2, db0, db1, db2,
                sg0, sg1, sg2, ss0, ss1, ss2):
    rows = [r0, r1, r2]
    ib = [ib0, ib1, ib2]
    db = [db0, db1, db2]
    semg = [sg0, sg1, sg2]
    sems = [ss0, ss1, ss2]

    cid = lax.axis_index("c")
    sid = lax.axis_index("s")
    wid = sid * _NC + cid
    eoff = wid * _EPT
    ntrips = (_NRCH // _NS) + jnp.where(sid < (_NRCH % _NS), 1, 0)

    def _zr(r, _):
        for c8 in range(F // 16):
            rows[0][r, pl.ds(c8 * 16, 16)] = jnp.zeros((16,), jnp.float32)
        return 0
    lax.fori_loop(0, _RCH, _zr, 0)

    def _zcp(k, _):
        pltpu.sync_copy(rows[0], acc_sh.at[pl.ds((sid + k * _NS) * _RCH, _RCH)])
        return 0
    lax.fori_loop(0, ntrips, _zcp, 0)

    pltpu.sync_copy(w_hbm.at[pl.ds(eoff, _EPT)], w_all)

    def _stage(gn):
        o = eoff + gn * _K
        pltpu.sync_copy(src_hbm.at[pl.ds(o, _SUP)], sbuf)
        pltpu.sync_copy(attr_hbm.at[pl.ds(o, _SUP)], abuf)
        pltpu.sync_copy(dst_hbm.at[pl.ds(o, _SUP)], dbuf)
    _stage(0)

    plsc.subcore_barrier()

    def _fire_g(gn, d):
        lg = lax.rem(gn, _GSUP) if not isinstance(gn, int) else gn % _GSUP
        if isinstance(gn, int):
            if gn > 0 and gn % _GSUP == 0:
                _stage(gn)
        else:
            @pl.when(jnp.logical_and(lg == 0, gn > 0))
            def _():
                _stage(gn)
        lo = lg * _K
        for j in range(_K // 16):
            t = pl.ds(j * 16, 16)
            u = pl.ds(lo + j * 16, 16)
            ib[d][t] = abuf[u] * N + sbuf[u]
            db[d][t] = dbuf[u]
        pltpu.async_copy(table_hbm.at[ib[d]], rows[d], semg[d])

    def _scale(g, d):
        def _sj(j, _):
            wv = w_all[pl.ds(g * _K + j * 16, 16)]
            for l in range(16):
                we = jnp.full((16,), wv[l], jnp.float32)
                e = j * 16 + l
                for c8 in range(F // 16):
                    s = pl.ds(c8 * 16, 16)
                    rows[d][e, s] = rows[d][e, s] * we
            return 0
        lax.fori_loop(0, _K // 16, _sj, 0)

    def _proc(g, d, pd):
        pltpu.make_async_copy(table_hbm.at[ib[d]], rows[d], semg[d]).wait()
        _scale(g, d)
        pltpu.async_copy(rows[d], acc_sh.at[db[d]], sems[d], add=True)

        @pl.when(jnp.bool_(g >= 1))
        def _():
            pltpu.make_async_copy(rows[pd], acc_sh.at[db[pd]], sems[pd]).wait()

        gn = g + _RDS - 1
        if isinstance(g, int):
            if gn < _NG:
                _fire_g(gn, pd)
        else:
            @pl.when(gn < _NG)
            def _():
                _fire_g(gn, pd)

    for d in range(_RDS - 1):
        _fire_g(d, d)

    def _loop(s_, _):
        for d in range(_RDS):
            _proc(s_ * _RDS + d, d, (d - 1) % _RDS)
        return 0
    lax.fori_loop(0, _NG // _RDS, _loop, 0)
    for g in range(_NG - _NG % _RDS, _NG):
        _proc(g, g % _RDS, (g - 1) % _RDS)
    dlast = (_NG - 1) % _RDS
    pltpu.make_async_copy(rows[dlast], acc_sh.at[db[dlast]], sems[dlast]).wait()

    plsc.subcore_barrier()

    def _wb(k, _):
        rr = (sid + k * _NS) * _RCH
        pltpu.sync_copy(acc_sh.at[pl.ds(rr, _RCH)], rows[0])
        pltpu.sync_copy(rows[0], out_hbm.at[cid, pl.ds(rr, _RCH)])
        return 0
    lax.fori_loop(0, ntrips, _wb, 0)


def _wmix_body(comp_ref, basis_ref, out_ref):
    out_ref[...] = jnp.dot(comp_ref[...], basis_ref[...],
                           preferred_element_type=jnp.float32)


def _wmix(comp, basis_flat):
    return pl.pallas_call(
        _wmix_body,
        out_shape=jax.ShapeDtypeStruct((R, F * F), jnp.float32),
    )(comp, basis_flat)


_BLK = 1000
_NBLK = N // _BLK
_LBLK = 10000
_LNB = N // _LBLK


def _layer0_body(x_ref, g_ref, b_ref, wr_ref, root_ref, bias_ref,
                 hall_ref, rrow_ref):
    r = pl.program_id(1)
    h = x_ref[...] * (g_ref[...] * _BN_S) + b_ref[...]
    hall_ref[...] = jnp.dot(h, wr_ref[0], preferred_element_type=jnp.float32)

    @pl.when(r == 0)
    def _():
        rrow_ref[...] = jnp.dot(
            h, root_ref[...], preferred_element_type=jnp.float32) + bias_ref[...]


def _layer0(x, g1, b1, wr, root, bias):
    return pl.pallas_call(
        _layer0_body,
        grid=(_LNB, R),
        in_specs=[
            pl.BlockSpec((_LBLK, F), lambda i, r: (i, 0)),
            pl.BlockSpec((1, F), lambda i, r: (0, 0)),
            pl.BlockSpec((1, F), lambda i, r: (0, 0)),
            pl.BlockSpec((1, F, F), lambda i, r: (r, 0, 0)),
            pl.BlockSpec((F, F), lambda i, r: (0, 0)),
            pl.BlockSpec((1, F), lambda i, r: (0, 0)),
        ],
        out_specs=[
            pl.BlockSpec((_LBLK, F), lambda i, r: (r * _LNB + i, 0)),
            pl.BlockSpec((_LBLK, F), lambda i, r: (i, 0)),
        ],
        out_shape=[
            jax.ShapeDtypeStruct((R * N, F), jnp.float32),
            jax.ShapeDtypeStruct((N, F), jnp.float32),
        ],
    )(x, g1, b1, wr, root, bias)


def _layer1_body(p_ref, rprev_ref, wr_ref, root_ref, bias_ref,
                 hall_ref, rrow_ref):
    r = pl.program_id(1)
    h = jnp.maximum(rprev_ref[...] + p_ref[0] + p_ref[1], 0.0)
    hall_ref[...] = jnp.dot(h, wr_ref[0], preferred_element_type=jnp.float32)

    @pl.when(r == 0)
    def _():
        rrow_ref[...] = jnp.dot(
            h, root_ref[...], preferred_element_type=jnp.float32) + bias_ref[...]


def _layer1(p, rprev, wr, root, bias):
    return pl.pallas_call(
        _layer1_body,
        grid=(_LNB, R),
        in_specs=[
            pl.BlockSpec((_NC, _LBLK, F), lambda i, r: (0, i, 0)),
            pl.BlockSpec((_LBLK, F), lambda i, r: (i, 0)),
            pl.BlockSpec((1, F, F), lambda i, r: (r, 0, 0)),
            pl.BlockSpec((F, F), lambda i, r: (0, 0)),
            pl.BlockSpec((1, F), lambda i, r: (0, 0)),
        ],
        out_specs=[
            pl.BlockSpec((_LBLK, F), lambda i, r: (r * _LNB + i, 0)),
            pl.BlockSpec((_LBLK, F), lambda i, r: (i, 0)),
        ],
        out_shape=[
            jax.ShapeDtypeStruct((R * N, F), jnp.float32),
            jax.ShapeDtypeStruct((N, F), jnp.float32),
        ],
    )(p, rprev, wr, root, bias)


def _head_body(p_ref, rprev_ref, batch_ref, g2_ref, b2_ref,
               w1_ref, c1_ref, w2_ref, c2_ref, out_ref, sums_ref, cnts_ref):
    i = pl.program_id(0)
    h = jnp.maximum(rprev_ref[...] + p_ref[0] + p_ref[1], 0.0)
    bat = batch_ref[0]
    gid = lax.broadcasted_iota(jnp.int32, (G, _BLK), 0)
    oneh = (gid == bat).astype(jnp.float32)
    s = jnp.dot(oneh, h, preferred_element_type=jnp.float32)
    c = jnp.dot(oneh, jnp.ones((_BLK, F), jnp.float32),
                preferred_element_type=jnp.float32)

    @pl.when(i == 0)
    def _():
        sums_ref[...] = s
        cnts_ref[...] = c

    @pl.when(i > 0)
    def _():
        sums_ref[...] += s
        cnts_ref[...] += c

    @pl.when(i == _NBLK - 1)
    def _():
        mean = sums_ref[...] / jnp.maximum(cnts_ref[...], 1.0)
        hb = mean * (g2_ref[...] * _BN_S) + b2_ref[...]
        z = jnp.maximum(jnp.dot(hb, w1_ref[...],
                                preferred_element_type=jnp.float32)
                        + c1_ref[...], 0.0)
        z = jnp.dot(z, w2_ref[...],
                    preferred_element_type=jnp.float32) + c2_ref[...]
        m = jnp.max(z, axis=-1, keepdims=True)
        out_ref[...] = z - m - jnp.log(
            jnp.sum(jnp.exp(z - m), axis=-1, keepdims=True))


def _head(p, rprev, batch3d, g2, b2, w1, c1, w2, c2):
    return pl.pallas_call(
        _head_body,
        grid=(_NBLK,),
        in_specs=[
            pl.BlockSpec((_NC, _BLK, F), lambda i: (0, i, 0)),
            pl.BlockSpec((_BLK, F), lambda i: (i, 0)),
            pl.BlockSpec((1, 1, _BLK), lambda i: (i, 0, 0)),
            pl.BlockSpec((1, F), lambda i: (0, 0)),
            pl.BlockSpec((1, F), lambda i: (0, 0)),
            pl.BlockSpec((F, F), lambda i: (0, 0)),
            pl.BlockSpec((1, F), lambda i: (0, 0)),
            pl.BlockSpec((F, C), lambda i: (0, 0)),
            pl.BlockSpec((1, C), lambda i: (0, 0)),
        ],
        out_specs=pl.BlockSpec((G, C), lambda i: (0, 0)),
        out_shape=jax.ShapeDtypeStruct((G, C), jnp.float32),
        scratch_shapes=[
            pltpu.VMEM((G, F), jnp.float32),
            pltpu.VMEM((G, F), jnp.float32),
        ],
    )(p, rprev, batch3d, g2, b2, w1, c1, w2, c2)


def kernel(x, edge_index, edge_attr, batch, bn1_g, bn1_b, basis0, comp0,
           root0, bias0, basis1, comp1, root1, bias1, bn2_g, bn2_b,
           fc1_W, fc1_b, fc2_W, fc2_b):
    src = edge_index[0]
    dst = edge_index[1]

    w = _sc_edge_weights(dst, edge_attr)

    wr0 = _wmix(comp0, basis0.reshape(NB, F * F)).reshape(R, F, F)
    wr1 = _wmix(comp1, basis1.reshape(NB, F * F)).reshape(R, F, F)

    hall0, rrow0 = _layer0(x, bn1_g.reshape(1, F), bn1_b.reshape(1, F),
                           wr0, root0, bias0.reshape(1, F))
    p0 = _sc_scatter(hall0, src, edge_attr, dst, w)

    hall1, rrow1 = _layer1(p0, rrow0, wr1, root1, bias1.reshape(1, F))
    p1 = _sc_scatter(hall1, src, edge_attr, dst, w)

    return _head(p1, rrow1, batch.reshape(_NBLK, 1, _BLK),
                 bn2_g.reshape(1, F), bn2_b.reshape(1, F),
                 fc1_W, fc1_b.reshape(1, F), fc2_W, fc2_b.reshape(1, C))
